# baseline (speedup 1.0000x reference)
import functools

import jax
import jax.numpy as jnp
from jax import lax
from jax.experimental import pallas as pl
from jax.experimental.pallas import tpu as pltpu
from jax.experimental.pallas import tpu_sc as plsc

N_ATOMS = 10000
N_EDGES = 320000
D = 128
DH = D // 2
N_DRUGS = 1000
NB = 16384
H0 = 1024
H1 = 512

NC = 2
NS = 16
NW = NC * NS
ECH = 128
ACH = 256
ANCH = N_EDGES // ACH
TPC = 78
TAIL_BASE = TPC * NS
TAIL = ANCH - TAIL_BASE
LCH = TPC + 1
NAP = 10240
STRIPE = NAP // NS
TPT = NB // NW


@functools.cache
def _mesh():
    return plsc.VectorSubcoreMesh(core_axis_name="c", subcore_axis_name="s",
                                  num_cores=NC, num_subcores=NS)


def _zero_vec():
    return jnp.zeros((16,), jnp.float32)


def _agg_body(src_h, dst_h, x_h, outs, src_v, dst_v, idxd_v, rows_v, hist_v,
              acc_sh, sems, *, first):
    c = lax.axis_index("c")
    s = lax.axis_index("s")
    wid = s * NC + c

    for arr_h, arr_v in ((src_h, src_v), (dst_h, dst_v)):
        pltpu.sync_copy(arr_h.at[pl.ds(s * TPC * ACH, TPC * ACH)],
                        arr_v.at[pl.ds(0, TPC * ACH)])

    @pl.when(s < TAIL)
    def _():
        for arr_h, arr_v in ((src_h, src_v), (dst_h, dst_v)):
            pltpu.sync_copy(arr_h.at[pl.ds((TAIL_BASE + s) * ACH, ACH)],
                            arr_v.at[pl.ds(TPC * ACH, ACH)])

    off = jnp.full((16,), c * N_ATOMS, jnp.int32)

    def shift(i, t):
        src_v[pl.ds(16 * i, 16)] = src_v[pl.ds(16 * i, 16)] + off
        return t
    lax.fori_loop(0, LCH * ACH // 16, shift, 0)

    def zrow(i, t):
        for k in range(DH // 16):
            rows_v[0, i, pl.ds(16 * k, 16)] = _zero_vec()
        return t
    lax.fori_loop(0, ACH, zrow, 0)
    if first:
        def zh(i, t):
            hist_v[pl.ds(16 * i, 16)] = _zero_vec()
            return t
        lax.fori_loop(0, N_ATOMS // 16, zh, 0)

    r0 = s * STRIPE
    for rr, ln in ((0, ACH), (ACH, ACH), (2 * ACH, ECH)):
        pltpu.sync_copy(rows_v.at[0, pl.ds(0, ln)],
                        acc_sh.at[pl.ds(r0 + rr, ln)])
    plsc.subcore_barrier()

    def fire_gather(k, p):
        return pltpu.async_copy(x_h.at[src_v.at[pl.ds(k * ACH, ACH)]],
                                rows_v.at[p], sems[p])

    def stage_dst(k, p):
        for q in range(ACH // 16):
            dv = dst_v[pl.ds(k * ACH + 16 * q, 16)]
            idxd_v[p, pl.ds(16 * q, 16)] = dv
            if first:
                plsc.addupdate_scatter(hist_v, [dv],
                                       jnp.ones((16,), jnp.float32))

    def fire_scatter(p):
        return pltpu.async_copy(rows_v.at[p], acc_sh.at[idxd_v.at[p]],
                                sems[p], add=True)

    def drain_scatter(p):
        pltpu.make_async_copy(rows_v.at[p], acc_sh.at[idxd_v.at[p]],
                              sems[p]).wait()

    def body(i, t):
        descs = []
        for p in range(2):
            @pl.when(i > 0)
            def _(p=p):
                drain_scatter(p)
            descs.append(fire_gather(2 * i + p, p))
            stage_dst(2 * i + p, p)
        for p in range(2):
            descs[p].wait()
            fire_scatter(p)
        return t

    lax.fori_loop(0, TPC // 2, body, 0)
    for p in range(2):
        drain_scatter(p)

    @pl.when(s < TAIL)
    def _():
        d = fire_gather(TPC, 0)
        stage_dst(TPC, 0)
        d.wait()
        pltpu.sync_copy(rows_v.at[0], acc_sh.at[idxd_v.at[0]], add=True)

    if first:
        agg_h, hist_h = outs
        pltpu.sync_copy(hist_v, hist_h.at[pl.ds(wid * N_ATOMS, N_ATOMS)])
    else:
        (agg_h,) = outs

    plsc.subcore_barrier()
    for rr, ln in ((0, ACH), (ACH, ACH), (2 * ACH, ECH)):
        pltpu.sync_copy(acc_sh.at[pl.ds(r0 + rr, ln)],
                        rows_v.at[0, pl.ds(0, ln)])
        pltpu.sync_copy(rows_v.at[0, pl.ds(0, ln)],
                        agg_h.at[c, pl.ds(r0 + rr, ln)])


def _agg_first_body(src_h, dst_h, x_h, agg_h, hist_h,
                    src_v, dst_v, idxd_v, rows_v, hist_v, acc_sh, sem0, sem1):
    _agg_body(src_h, dst_h, x_h, (agg_h, hist_h),
              src_v, dst_v, idxd_v, rows_v, hist_v, acc_sh,
              (sem0, sem1), first=True)


def _agg_second_body(src_h, dst_h, x_h, agg_h,
                     src_v, dst_v, idxd_v, rows_v, hist_v, acc_sh, sem0, sem1):
    _agg_body(src_h, dst_h, x_h, (agg_h,),
              src_v, dst_v, idxd_v, rows_v, hist_v, acc_sh,
              (sem0, sem1), first=False)


def _agg_scratch():
    return [
        pltpu.VMEM((LCH * ACH,), jnp.int32),
        pltpu.VMEM((LCH * ACH,), jnp.int32),
        pltpu.VMEM((2, ACH), jnp.int32),
        pltpu.VMEM((2, ACH, DH), jnp.float32),
        pltpu.VMEM((N_ATOMS,), jnp.float32),
        pltpu.VMEM_SHARED((NAP, DH), jnp.float32),
        pltpu.SemaphoreType.DMA,
        pltpu.SemaphoreType.DMA,
    ]


_sc_params = pltpu.CompilerParams(needs_layout_passes=False,
                                  use_tc_tiling_on_sc=False)


@functools.cache
def _agg_first():
    return pl.kernel(
        _agg_first_body, mesh=_mesh(), scratch_types=_agg_scratch(),
        compiler_params=_sc_params,
        out_type=[
            jax.ShapeDtypeStruct((NC, NAP, DH), jnp.float32),
            jax.ShapeDtypeStruct((NW * N_ATOMS,), jnp.float32),
        ])


@functools.cache
def _agg_second():
    return pl.kernel(
        _agg_second_body, mesh=_mesh(), scratch_types=_agg_scratch(),
        compiler_params=_sc_params,
        out_type=jax.ShapeDtypeStruct((NC, NAP, DH), jnp.float32))


def _trip_gather_body(i0_h, i1_h, i2_h, d_h, clf_h, da_h, db_h, cl_h,
                      idx_v, rows_v, sem0, sem1, sem2, sem3):
    c = lax.axis_index("c")
    s = lax.axis_index("s")
    wid = s * NC + c
    base = wid * TPT
    sems = (sem0, sem1, sem2, sem3)
    for j, src_h in enumerate((i0_h, i1_h, i2_h)):
        pltpu.sync_copy(src_h.at[pl.ds(base, TPT)],
                        idx_v.at[pl.ds(j * TPT, TPT)])

    jobs = []
    for j, (tbl_h, out_h) in enumerate(((d_h, da_h), (d_h, db_h),
                                        (clf_h, cl_h))):
        for q in range(TPT // ECH):
            jobs.append((tbl_h, out_h, j * TPT + q * ECH, base + q * ECH))

    def fire_g(job, p):
        tbl_h, _, ioff, _ = job
        return pltpu.async_copy(tbl_h.at[idx_v.at[pl.ds(ioff, ECH)]],
                                rows_v.at[p], sems[p])

    def fire_w(job, p):
        _, out_h, _, ooff = job
        return pltpu.async_copy(rows_v.at[p], out_h.at[pl.ds(ooff, ECH)],
                                sems[p])

    def drain_w(job, p):
        _, out_h, _, ooff = job
        pltpu.make_async_copy(rows_v.at[p], out_h.at[pl.ds(ooff, ECH)],
                              sems[p]).wait()

    ngroups = len(jobs) // 4
    for g in range(ngroups):
        descs = []
        for p in range(4):
            if g > 0:
                drain_w(jobs[4 * (g - 1) + p], p)
            descs.append(fire_g(jobs[4 * g + p], p))
        for p in range(4):
            descs[p].wait()
            fire_w(jobs[4 * g + p], p)
    for p in range(4):
        drain_w(jobs[4 * (ngroups - 1) + p], p)


@functools.cache
def _trip_gather():
    return pl.kernel(
        _trip_gather_body, mesh=_mesh(),
        compiler_params=_sc_params,
        scratch_types=[
            pltpu.VMEM((3 * TPT,), jnp.int32),
            pltpu.VMEM((4, ECH, D), jnp.float32),
            pltpu.SemaphoreType.DMA,
            pltpu.SemaphoreType.DMA,
            pltpu.SemaphoreType.DMA,
            pltpu.SemaphoreType.DMA,
        ],
        out_type=[
            jax.ShapeDtypeStruct((NB, D), jnp.float32),
            jax.ShapeDtypeStruct((NB, D), jnp.float32),
            jax.ShapeDtypeStruct((NB, D), jnp.float32),
        ])


def _ln_rows(x, g, b, eps=1e-5):
    mu = jnp.mean(x, axis=-1, keepdims=True)
    var = jnp.mean((x - mu) ** 2, axis=-1, keepdims=True)
    return (x - mu) * jax.lax.rsqrt(var + eps) * g + b


def _deg_body(hist_ref, out_ref):
    deg = jnp.sum(hist_ref[...], axis=0, keepdims=True)
    out_ref[...] = 2.0 / jnp.maximum(deg, 2.0)


def _rdeg(hist):
    r = pl.pallas_call(
        _deg_body,
        out_shape=jax.ShapeDtypeStruct((1, N_ATOMS), jnp.float32),
    )(hist)
    return r.reshape(N_ATOMS, 1)


_RB0 = 2000


def _sage0_body(x_ref, agg_ref, rd_ref, ws_ref, wna_ref, wnb_ref, b_ref,
                g_ref, bl_ref, out_ref):
    rd = rd_ref[...]
    y = (jnp.dot(x_ref[...], ws_ref[...], preferred_element_type=jnp.float32)
         + jnp.dot(agg_ref[0] * rd, wna_ref[...],
                   preferred_element_type=jnp.float32)
         + jnp.dot(agg_ref[1] * rd, wnb_ref[...],
                   preferred_element_type=jnp.float32)
         + b_ref[...])
    h = jnp.maximum(_ln_rows(y, g_ref[...], bl_ref[...]), 0.0)
    out_ref[0] = h[:, :DH]
    out_ref[1] = h[:, DH:]


def _sage0(x, agg2, rdeg, Ws, Wna, Wnb, b, g, bl):
    n = N_ATOMS // _RB0
    return pl.pallas_call(
        _sage0_body,
        grid=(n,),
        in_specs=[
            pl.BlockSpec((_RB0, D), lambda i: (i, 0)),
            pl.BlockSpec((NC, _RB0, DH), lambda i: (0, i, 0)),
            pl.BlockSpec((_RB0, 1), lambda i: (i, 0)),
            pl.BlockSpec((D, D), lambda i: (0, 0)),
            pl.BlockSpec((DH, D), lambda i: (0, 0)),
            pl.BlockSpec((DH, D), lambda i: (0, 0)),
            pl.BlockSpec((1, D), lambda i: (0, 0)),
            pl.BlockSpec((1, D), lambda i: (0, 0)),
            pl.BlockSpec((1, D), lambda i: (0, 0)),
        ],
        out_specs=pl.BlockSpec((NC, _RB0, DH), lambda i: (0, i, 0)),
        out_shape=jax.ShapeDtypeStruct((NC, N_ATOMS, DH), jnp.float32),
    )(x, agg2, rdeg, Ws, Wna, Wnb, b, g, bl)


def _sage1_body(h_ref, agg_ref, rd_ref, wsa_ref, wsb_ref, wna_ref, wnb_ref,
                b_ref, wo_ref, bo_ref, g_ref, bl_ref, out_ref):
    rd = rd_ref[...]
    y = (jnp.dot(h_ref[0], wsa_ref[...], preferred_element_type=jnp.float32)
         + jnp.dot(h_ref[1], wsb_ref[...], preferred_element_type=jnp.float32)
         + jnp.dot(agg_ref[0] * rd, wna_ref[...],
                   preferred_element_type=jnp.float32)
         + jnp.dot(agg_ref[1] * rd, wnb_ref[...],
                   preferred_element_type=jnp.float32)
         + b_ref[...])
    pooled = jnp.max(y.reshape(_RB0 // 10, 10, D), axis=1)
    dd = jnp.maximum(
        jnp.dot(pooled, wo_ref[...], preferred_element_type=jnp.float32)
        + bo_ref[...], 0.0)
    out_ref[...] = _ln_rows(dd, g_ref[...], bl_ref[...])


def _sage1(h2, agg2, rdeg, Wsa, Wsb, Wna, Wnb, b, Wo, bo, g, bl):
    n = N_ATOMS // _RB0
    return pl.pallas_call(
        _sage1_body,
        grid=(n,),
        in_specs=[
            pl.BlockSpec((NC, _RB0, DH), lambda i: (0, i, 0)),
            pl.BlockSpec((NC, _RB0, DH), lambda i: (0, i, 0)),
            pl.BlockSpec((_RB0, 1), lambda i: (i, 0)),
            pl.BlockSpec((DH, D), lambda i: (0, 0)),
            pl.BlockSpec((DH, D), lambda i: (0, 0)),
            pl.BlockSpec((DH, D), lambda i: (0, 0)),
            pl.BlockSpec((DH, D), lambda i: (0, 0)),
            pl.BlockSpec((1, D), lambda i: (0, 0)),
            pl.BlockSpec((D, D), lambda i: (0, 0)),
            pl.BlockSpec((1, D), lambda i: (0, 0)),
            pl.BlockSpec((1, D), lambda i: (0, 0)),
            pl.BlockSpec((1, D), lambda i: (0, 0)),
        ],
        out_specs=pl.BlockSpec((_RB0 // 10, D), lambda i: (i, 0)),
        out_shape=jax.ShapeDtypeStruct((N_DRUGS, D), jnp.float32),
    )(h2, agg2, rdeg, Wsa, Wsb, Wna, Wnb, b, Wo, bo, g, bl)


_RBF = 2048


def _mlp_body(da_ref, db_ref, cl_ref, gcl_ref, bcl_ref,
              w0a_ref, w0b_ref, w0c_ref, bt0_ref, gt_ref, btl_ref,
              wt1_ref, bt1_ref, wp_ref, bp_ref, out_ref):
    cl = _ln_rows(cl_ref[...], gcl_ref[...], bcl_ref[...])
    t = (jnp.dot(da_ref[...], w0a_ref[...], preferred_element_type=jnp.float32)
         + jnp.dot(db_ref[...], w0b_ref[...], preferred_element_type=jnp.float32)
         + jnp.dot(cl, w0c_ref[...], preferred_element_type=jnp.float32)
         + bt0_ref[...])
    h = _ln_rows(jnp.maximum(t, 0.0), gt_ref[...], btl_ref[...])
    h2 = jnp.maximum(
        jnp.dot(h, wt1_ref[...], preferred_element_type=jnp.float32)
        + bt1_ref[...], 0.0)
    out_ref[...] = (jnp.dot(h2, wp_ref[...],
                            preferred_element_type=jnp.float32) + bp_ref[...])


def _mlp(da, db, cl, gcl, bcl, W0a, W0b, W0c, bt0, gt, btl, Wt1, bt1, Wp, bp):
    n = NB // _RBF
    row = lambda i: (i, 0)
    fix = lambda i: (0, 0)
    return pl.pallas_call(
        _mlp_body,
        grid=(n,),
        in_specs=[
            pl.BlockSpec((_RBF, D), row),
            pl.BlockSpec((_RBF, D), row),
            pl.BlockSpec((_RBF, D), row),
            pl.BlockSpec((1, D), fix),
            pl.BlockSpec((1, D), fix),
            pl.BlockSpec((D, H0), fix),
            pl.BlockSpec((D, H0), fix),
            pl.BlockSpec((D, H0), fix),
            pl.BlockSpec((1, H0), fix),
            pl.BlockSpec((1, H0), fix),
            pl.BlockSpec((1, H0), fix),
            pl.BlockSpec((H0, H1), fix),
            pl.BlockSpec((1, H1), fix),
            pl.BlockSpec((H1, 1), fix),
            pl.BlockSpec((1, 1), fix),
        ],
        out_specs=pl.BlockSpec((_RBF, 1), row),
        out_shape=jax.ShapeDtypeStruct((NB, 1), jnp.float32),
    )(da, db, cl, gcl, bcl, W0a, W0b, W0c, bt0, gt, btl, Wt1, bt1, Wp, bp)


def kernel(indices, atom_features, edge_index, batch, Ws0, Wn0, b0, g_ln0,
           b_ln0, Ws1, Wn1, b1, Wout, b_out, g_dn, b_dn, cl_feat, g_cl, b_cl,
           W_t0, b_t0, g_t, b_tln, W_t1, b_t1, W_p, b_p):
    src = edge_index[0]
    dst = edge_index[1]
    idx0 = indices[:, 0]
    idx1 = indices[:, 1]
    idx2 = indices[:, 2]
    r1 = lambda v: v.reshape(1, -1)

    xs0 = jnp.concatenate([atom_features[:, :DH], atom_features[:, DH:]],
                          axis=0)
    agg0, hist = _agg_first()(src, dst, xs0)
    rdeg = _rdeg(hist.reshape(NW, N_ATOMS))
    h2 = _sage0(atom_features, agg0, rdeg, Ws0, Wn0[:DH], Wn0[DH:], r1(b0),
                r1(g_ln0), r1(b_ln0))
    agg1 = _agg_second()(src, dst, h2.reshape(NC * N_ATOMS, DH))
    d = _sage1(h2, agg1, rdeg, Ws1[:DH], Ws1[DH:], Wn1[:DH], Wn1[DH:],
               r1(b1), Wout, r1(b_out), r1(g_dn), r1(b_dn))
    da, db, cl_rows = _trip_gather()(idx0, idx1, idx2, d, cl_feat)
    out = _mlp(da, db, cl_rows, r1(g_cl), r1(b_cl),
               W_t0[:D], W_t0[D:2 * D], W_t0[2 * D:], r1(b_t0), r1(g_t),
               r1(b_tln), W_t1, r1(b_t1), W_p, r1(b_p))
    return out[:, 0]


# speedup vs baseline: 1.1637x; 1.1637x over previous
import functools

import jax
import jax.numpy as jnp
from jax import lax
from jax.experimental import pallas as pl
from jax.experimental.pallas import tpu as pltpu
from jax.experimental.pallas import tpu_sc as plsc

N_ATOMS = 10000
N_EDGES = 320000
D = 128
DH = D // 2
N_DRUGS = 1000
NB = 16384
H0 = 1024
H1 = 512

NC = 2
NS = 16
NW = NC * NS
ECH = 128
NCHUNKS = N_EDGES // ECH
GSZ = 3
TPC = 156
NGRP = TPC // GSZ
TAIL_BASE = TPC * NS
TAIL = NCHUNKS - TAIL_BASE
NAP = 10240
STRIPE = NAP // NS
TPT = NB // NW


@functools.cache
def _mesh():
    return plsc.VectorSubcoreMesh(core_axis_name="c", subcore_axis_name="s",
                                  num_cores=NC, num_subcores=NS)


def _zero_vec():
    return jnp.zeros((16,), jnp.float32)


def _agg_body(src_h, dst_h, x_h, outs, srcb_v, dstb_v, idxd_v, rows_v, hist_v,
              acc_sh, sems, isems, *, first):
    c = lax.axis_index("c")
    s = lax.axis_index("s")
    wid = s * NC + c
    cbase = s * TPC
    GW = GSZ * ECH
    off = jnp.full((16,), c * N_ATOMS, jnp.int32)

    def fire_idx(g, b):
        for arr_h, arr_v in ((src_h, srcb_v), (dst_h, dstb_v)):
            pltpu.async_copy(arr_h.at[pl.ds((cbase + g * GSZ) * ECH, GW)],
                             arr_v.at[pl.ds(b * GW, GW)], isems[b])

    def drain_idx(b):
        for arr_h, arr_v in ((src_h, srcb_v), (dst_h, dstb_v)):
            pltpu.make_async_copy(arr_h.at[pl.ds(0, GW)],
                                  arr_v.at[pl.ds(b * GW, GW)],
                                  isems[b]).wait()

    def zrow(i, t):
        for k in range(DH // 16):
            rows_v[0, i, pl.ds(16 * k, 16)] = _zero_vec()
        return t
    lax.fori_loop(0, ECH, zrow, 0)
    if first:
        def zh(i, t):
            hist_v[pl.ds(16 * i, 16)] = _zero_vec()
            return t
        lax.fori_loop(0, N_ATOMS // 16, zh, 0)

    r0 = s * STRIPE
    for k in range(5):
        pltpu.sync_copy(rows_v.at[0], acc_sh.at[pl.ds(r0 + k * ECH, ECH)])
    fire_idx(0, 0)
    drain_idx(0)
    plsc.subcore_barrier()

    def fire_gather(b, p, slot):
        return pltpu.async_copy(
            x_h.at[srcb_v.at[pl.ds(b * GW + p * ECH, ECH)]],
            rows_v.at[slot], sems[slot])

    def stage_idx(b, p, slot):
        for q in range(ECH // 16):
            w = b * GW + p * ECH + 16 * q
            srcb_v[pl.ds(w, 16)] = srcb_v[pl.ds(w, 16)] + off
            dv = dstb_v[pl.ds(w, 16)]
            idxd_v[slot, pl.ds(16 * q, 16)] = dv
            if first:
                plsc.addupdate_scatter(hist_v, [dv],
                                       jnp.ones((16,), jnp.float32))

    def fire_scatter(slot):
        return pltpu.async_copy(rows_v.at[slot], acc_sh.at[idxd_v.at[slot]],
                                sems[slot], add=True)

    def drain_scatter(slot):
        pltpu.make_async_copy(rows_v.at[slot], acc_sh.at[idxd_v.at[slot]],
                              sems[slot]).wait()

    def process_group(b, s0, cond):
        descs = []
        for p in range(GSZ):
            @pl.when(cond)
            def _(p=p):
                drain_scatter(s0 + p)
            stage_idx(b, p, s0 + p)
            descs.append(fire_gather(b, p, s0 + p))
        for p in range(GSZ):
            descs[p].wait()
            fire_scatter(s0 + p)

    def body(i, t):
        fire_idx(2 * i + 1, 1)
        process_group(0, 0, i > 0)
        drain_idx(1)

        @pl.when(i < NGRP // 2 - 1)
        def _():
            fire_idx(2 * i + 2, 0)
        process_group(1, GSZ, i > 0)

        @pl.when(i < NGRP // 2 - 1)
        def _():
            drain_idx(0)
        return t

    lax.fori_loop(0, NGRP // 2, body, 0)
    for slot in range(2 * GSZ):
        drain_scatter(slot)

    @pl.when(s < TAIL)
    def _():
        for arr_h, arr_v in ((src_h, srcb_v), (dst_h, dstb_v)):
            pltpu.sync_copy(arr_h.at[pl.ds((TAIL_BASE + s) * ECH, ECH)],
                            arr_v.at[pl.ds(0, ECH)])
        stage_idx(0, 0, 0)
        fire_gather(0, 0, 0).wait()
        pltpu.sync_copy(rows_v.at[0], acc_sh.at[idxd_v.at[0]], add=True)

    if first:
        agg_h, hist_h = outs
        pltpu.sync_copy(hist_v, hist_h.at[pl.ds(wid * N_ATOMS, N_ATOMS)])
    else:
        (agg_h,) = outs

    plsc.subcore_barrier()
    for k in range(5):
        rr = r0 + k * ECH
        pltpu.sync_copy(acc_sh.at[pl.ds(rr, ECH)], rows_v.at[0])
        pltpu.sync_copy(rows_v.at[0], agg_h.at[c, pl.ds(rr, ECH)])


def _agg_first_body(src_h, dst_h, x_h, agg_h, hist_h,
                    srcb_v, dstb_v, idxd_v, rows_v, hist_v, acc_sh,
                    s0, s1, s2, s3, s4, s5, i0, i1):
    _agg_body(src_h, dst_h, x_h, (agg_h, hist_h),
              srcb_v, dstb_v, idxd_v, rows_v, hist_v, acc_sh,
              (s0, s1, s2, s3, s4, s5), (i0, i1), first=True)


def _agg_second_body(src_h, dst_h, x_h, agg_h,
                     srcb_v, dstb_v, idxd_v, rows_v, hist_v, acc_sh,
                     s0, s1, s2, s3, s4, s5, i0, i1):
    _agg_body(src_h, dst_h, x_h, (agg_h,),
              srcb_v, dstb_v, idxd_v, rows_v, hist_v, acc_sh,
              (s0, s1, s2, s3, s4, s5), (i0, i1), first=False)


def _agg_scratch():
    return [
        pltpu.VMEM((2 * GSZ * ECH,), jnp.int32),
        pltpu.VMEM((2 * GSZ * ECH,), jnp.int32),
        pltpu.VMEM((2 * GSZ, ECH), jnp.int32),
        pltpu.VMEM((2 * GSZ, ECH, DH), jnp.float32),
        pltpu.VMEM((N_ATOMS,), jnp.float32),
        pltpu.VMEM_SHARED((NAP, DH), jnp.float32),
    ] + [pltpu.SemaphoreType.DMA] * 8


_sc_params = pltpu.CompilerParams(needs_layout_passes=False,
                                  use_tc_tiling_on_sc=False)


@functools.cache
def _agg_first():
    return pl.kernel(
        _agg_first_body, mesh=_mesh(), scratch_types=_agg_scratch(),
        compiler_params=_sc_params,
        out_type=[
            jax.ShapeDtypeStruct((NC, NAP, DH), jnp.float32),
            jax.ShapeDtypeStruct((NW * N_ATOMS,), jnp.float32),
        ])


@functools.cache
def _agg_second():
    return pl.kernel(
        _agg_second_body, mesh=_mesh(), scratch_types=_agg_scratch(),
        compiler_params=_sc_params,
        out_type=jax.ShapeDtypeStruct((NC, NAP, DH), jnp.float32))


def _trip_gather_body(i0_h, i1_h, i2_h, d_h, clf_h, da_h, db_h, cl_h,
                      idx_v, rows_v, sem0, sem1, sem2, sem3):
    c = lax.axis_index("c")
    s = lax.axis_index("s")
    wid = s * NC + c
    base = wid * TPT
    sems = (sem0, sem1, sem2, sem3)
    for j, src_h in enumerate((i0_h, i1_h, i2_h)):
        pltpu.sync_copy(src_h.at[pl.ds(base, TPT)],
                        idx_v.at[pl.ds(j * TPT, TPT)])

    jobs = []
    for j, (tbl_h, out_h) in enumerate(((d_h, da_h), (d_h, db_h),
                                        (clf_h, cl_h))):
        for q in range(TPT // ECH):
            jobs.append((tbl_h, out_h, j * TPT + q * ECH, base + q * ECH))

    def fire_g(job, p):
        tbl_h, _, ioff, _ = job
        return pltpu.async_copy(tbl_h.at[idx_v.at[pl.ds(ioff, ECH)]],
                                rows_v.at[p], sems[p])

    def fire_w(job, p):
        _, out_h, _, ooff = job
        return pltpu.async_copy(rows_v.at[p], out_h.at[pl.ds(ooff, ECH)],
                                sems[p])

    def drain_w(job, p):
        _, out_h, _, ooff = job
        pltpu.make_async_copy(rows_v.at[p], out_h.at[pl.ds(ooff, ECH)],
                              sems[p]).wait()

    ngroups = len(jobs) // 4
    for g in range(ngroups):
        descs = []
        for p in range(4):
            if g > 0:
                drain_w(jobs[4 * (g - 1) + p], p)
            descs.append(fire_g(jobs[4 * g + p], p))
        for p in range(4):
            descs[p].wait()
            fire_w(jobs[4 * g + p], p)
    for p in range(4):
        drain_w(jobs[4 * (ngroups - 1) + p], p)


@functools.cache
def _trip_gather():
    return pl.kernel(
        _trip_gather_body, mesh=_mesh(),
        compiler_params=_sc_params,
        scratch_types=[
            pltpu.VMEM((3 * TPT,), jnp.int32),
            pltpu.VMEM((4, ECH, D), jnp.float32),
            pltpu.SemaphoreType.DMA,
            pltpu.SemaphoreType.DMA,
            pltpu.SemaphoreType.DMA,
            pltpu.SemaphoreType.DMA,
        ],
        out_type=[
            jax.ShapeDtypeStruct((NB, D), jnp.float32),
            jax.ShapeDtypeStruct((NB, D), jnp.float32),
            jax.ShapeDtypeStruct((NB, D), jnp.float32),
        ])


def _ln_rows(x, g, b, eps=1e-5):
    mu = jnp.mean(x, axis=-1, keepdims=True)
    var = jnp.mean((x - mu) ** 2, axis=-1, keepdims=True)
    return (x - mu) * jax.lax.rsqrt(var + eps) * g + b


def _deg_body(hist_ref, out_ref):
    deg = jnp.sum(hist_ref[...], axis=0, keepdims=True)
    out_ref[...] = 2.0 / jnp.maximum(deg, 2.0)


def _rdeg(hist):
    r = pl.pallas_call(
        _deg_body,
        out_shape=jax.ShapeDtypeStruct((1, N_ATOMS), jnp.float32),
    )(hist)
    return r.reshape(N_ATOMS, 1)


_RB0 = 2000


def _sage0_body(x_ref, agg_ref, rd_ref, ws_ref, wna_ref, wnb_ref, b_ref,
                g_ref, bl_ref, out_ref):
    rd = rd_ref[...]
    y = (jnp.dot(x_ref[...], ws_ref[...], preferred_element_type=jnp.float32)
         + jnp.dot(agg_ref[0] * rd, wna_ref[...],
                   preferred_element_type=jnp.float32)
         + jnp.dot(agg_ref[1] * rd, wnb_ref[...],
                   preferred_element_type=jnp.float32)
         + b_ref[...])
    h = jnp.maximum(_ln_rows(y, g_ref[...], bl_ref[...]), 0.0)
    out_ref[0] = h[:, :DH]
    out_ref[1] = h[:, DH:]


def _sage0(x, agg2, rdeg, Ws, Wna, Wnb, b, g, bl):
    n = N_ATOMS // _RB0
    return pl.pallas_call(
        _sage0_body,
        grid=(n,),
        in_specs=[
            pl.BlockSpec((_RB0, D), lambda i: (i, 0)),
            pl.BlockSpec((NC, _RB0, DH), lambda i: (0, i, 0)),
            pl.BlockSpec((_RB0, 1), lambda i: (i, 0)),
            pl.BlockSpec((D, D), lambda i: (0, 0)),
            pl.BlockSpec((DH, D), lambda i: (0, 0)),
            pl.BlockSpec((DH, D), lambda i: (0, 0)),
            pl.BlockSpec((1, D), lambda i: (0, 0)),
            pl.BlockSpec((1, D), lambda i: (0, 0)),
            pl.BlockSpec((1, D), lambda i: (0, 0)),
        ],
        out_specs=pl.BlockSpec((NC, _RB0, DH), lambda i: (0, i, 0)),
        out_shape=jax.ShapeDtypeStruct((NC, N_ATOMS, DH), jnp.float32),
    )(x, agg2, rdeg, Ws, Wna, Wnb, b, g, bl)


def _sage1_body(h_ref, agg_ref, rd_ref, wsa_ref, wsb_ref, wna_ref, wnb_ref,
                b_ref, wo_ref, bo_ref, g_ref, bl_ref, out_ref):
    rd = rd_ref[...]
    y = (jnp.dot(h_ref[0], wsa_ref[...], preferred_element_type=jnp.float32)
         + jnp.dot(h_ref[1], wsb_ref[...], preferred_element_type=jnp.float32)
         + jnp.dot(agg_ref[0] * rd, wna_ref[...],
                   preferred_element_type=jnp.float32)
         + jnp.dot(agg_ref[1] * rd, wnb_ref[...],
                   preferred_element_type=jnp.float32)
         + b_ref[...])
    pooled = jnp.max(y.reshape(_RB0 // 10, 10, D), axis=1)
    dd = jnp.maximum(
        jnp.dot(pooled, wo_ref[...], preferred_element_type=jnp.float32)
        + bo_ref[...], 0.0)
    out_ref[...] = _ln_rows(dd, g_ref[...], bl_ref[...])


def _sage1(h2, agg2, rdeg, Wsa, Wsb, Wna, Wnb, b, Wo, bo, g, bl):
    n = N_ATOMS // _RB0
    return pl.pallas_call(
        _sage1_body,
        grid=(n,),
        in_specs=[
            pl.BlockSpec((NC, _RB0, DH), lambda i: (0, i, 0)),
            pl.BlockSpec((NC, _RB0, DH), lambda i: (0, i, 0)),
            pl.BlockSpec((_RB0, 1), lambda i: (i, 0)),
            pl.BlockSpec((DH, D), lambda i: (0, 0)),
            pl.BlockSpec((DH, D), lambda i: (0, 0)),
            pl.BlockSpec((DH, D), lambda i: (0, 0)),
            pl.BlockSpec((DH, D), lambda i: (0, 0)),
            pl.BlockSpec((1, D), lambda i: (0, 0)),
            pl.BlockSpec((D, D), lambda i: (0, 0)),
            pl.BlockSpec((1, D), lambda i: (0, 0)),
            pl.BlockSpec((1, D), lambda i: (0, 0)),
            pl.BlockSpec((1, D), lambda i: (0, 0)),
        ],
        out_specs=pl.BlockSpec((_RB0 // 10, D), lambda i: (i, 0)),
        out_shape=jax.ShapeDtypeStruct((N_DRUGS, D), jnp.float32),
    )(h2, agg2, rdeg, Wsa, Wsb, Wna, Wnb, b, Wo, bo, g, bl)


_RBF = 2048


def _mlp_body(da_ref, db_ref, cl_ref, gcl_ref, bcl_ref,
              w0a_ref, w0b_ref, w0c_ref, bt0_ref, gt_ref, btl_ref,
              wt1_ref, bt1_ref, wp_ref, bp_ref, out_ref):
    cl = _ln_rows(cl_ref[...], gcl_ref[...], bcl_ref[...])
    t = (jnp.dot(da_ref[...], w0a_ref[...], preferred_element_type=jnp.float32)
         + jnp.dot(db_ref[...], w0b_ref[...], preferred_element_type=jnp.float32)
         + jnp.dot(cl, w0c_ref[...], preferred_element_type=jnp.float32)
         + bt0_ref[...])
    h = _ln_rows(jnp.maximum(t, 0.0), gt_ref[...], btl_ref[...])
    h2 = jnp.maximum(
        jnp.dot(h, wt1_ref[...], preferred_element_type=jnp.float32)
        + bt1_ref[...], 0.0)
    out_ref[...] = (jnp.dot(h2, wp_ref[...],
                            preferred_element_type=jnp.float32) + bp_ref[...])


def _mlp(da, db, cl, gcl, bcl, W0a, W0b, W0c, bt0, gt, btl, Wt1, bt1, Wp, bp):
    n = NB // _RBF
    row = lambda i: (i, 0)
    fix = lambda i: (0, 0)
    return pl.pallas_call(
        _mlp_body,
        grid=(n,),
        in_specs=[
            pl.BlockSpec((_RBF, D), row),
            pl.BlockSpec((_RBF, D), row),
            pl.BlockSpec((_RBF, D), row),
            pl.BlockSpec((1, D), fix),
            pl.BlockSpec((1, D), fix),
            pl.BlockSpec((D, H0), fix),
            pl.BlockSpec((D, H0), fix),
            pl.BlockSpec((D, H0), fix),
            pl.BlockSpec((1, H0), fix),
            pl.BlockSpec((1, H0), fix),
            pl.BlockSpec((1, H0), fix),
            pl.BlockSpec((H0, H1), fix),
            pl.BlockSpec((1, H1), fix),
            pl.BlockSpec((H1, 1), fix),
            pl.BlockSpec((1, 1), fix),
        ],
        out_specs=pl.BlockSpec((_RBF, 1), row),
        out_shape=jax.ShapeDtypeStruct((NB, 1), jnp.float32),
    )(da, db, cl, gcl, bcl, W0a, W0b, W0c, bt0, gt, btl, Wt1, bt1, Wp, bp)


def kernel(indices, atom_features, edge_index, batch, Ws0, Wn0, b0, g_ln0,
           b_ln0, Ws1, Wn1, b1, Wout, b_out, g_dn, b_dn, cl_feat, g_cl, b_cl,
           W_t0, b_t0, g_t, b_tln, W_t1, b_t1, W_p, b_p):
    src = edge_index[0]
    dst = edge_index[1]
    idx0 = indices[:, 0]
    idx1 = indices[:, 1]
    idx2 = indices[:, 2]
    r1 = lambda v: v.reshape(1, -1)

    xs0 = jnp.concatenate([atom_features[:, :DH], atom_features[:, DH:]],
                          axis=0)
    agg0, hist = _agg_first()(src, dst, xs0)
    rdeg = _rdeg(hist.reshape(NW, N_ATOMS))
    h2 = _sage0(atom_features, agg0, rdeg, Ws0, Wn0[:DH], Wn0[DH:], r1(b0),
                r1(g_ln0), r1(b_ln0))
    agg1 = _agg_second()(src, dst, h2.reshape(NC * N_ATOMS, DH))
    d = _sage1(h2, agg1, rdeg, Ws1[:DH], Ws1[DH:], Wn1[:DH], Wn1[DH:],
               r1(b1), Wout, r1(b_out), r1(g_dn), r1(b_dn))
    da, db, cl_rows = _trip_gather()(idx0, idx1, idx2, d, cl_feat)
    out = _mlp(da, db, cl_rows, r1(g_cl), r1(b_cl),
               W_t0[:D], W_t0[D:2 * D], W_t0[2 * D:], r1(b_t0), r1(g_t),
               r1(b_tln), W_t1, r1(b_t1), W_p, r1(b_p))
    return out[:, 0]


# speedup vs baseline: 1.2511x; 1.0751x over previous
import functools

import jax
import jax.numpy as jnp
from jax import lax
from jax.experimental import pallas as pl
from jax.experimental.pallas import tpu as pltpu
from jax.experimental.pallas import tpu_sc as plsc

N_ATOMS = 10000
N_EDGES = 320000
D = 128
DH = D // 2
N_DRUGS = 1000
NB = 16384
H0 = 1024
H1 = 512

NC = 2
NS = 16
NW = NC * NS
ECH = 128
NCHUNKS = N_EDGES // ECH
GSZ = 3
TPC = 156
NGRP = TPC // GSZ
TAIL_BASE = TPC * NS
TAIL = NCHUNKS - TAIL_BASE
NAP = 10240
STRIPE = NAP // NS
TPT = NB // NW


@functools.cache
def _mesh():
    return plsc.VectorSubcoreMesh(core_axis_name="c", subcore_axis_name="s",
                                  num_cores=NC, num_subcores=NS)


def _zero_vec():
    return jnp.zeros((16,), jnp.float32)


def _agg_body(edge_h, x_h, outs, srcb_v, dstb_v, idxd_v, rows_v, hist_v,
              acc_sh, sems, isems, *, first):
    c = lax.axis_index("c")
    s = lax.axis_index("s")
    wid = s * NC + c
    cbase = s * TPC
    GW = GSZ * ECH
    off = jnp.full((16,), c if first else c * N_ATOMS, jnp.int32)

    def fire_idx(g, b):
        for a, arr_v in ((0, srcb_v), (1, dstb_v)):
            pltpu.async_copy(edge_h.at[a, pl.ds((cbase + g * GSZ) * ECH, GW)],
                             arr_v.at[pl.ds(b * GW, GW)], isems[b])

    def drain_idx(b):
        for a, arr_v in ((0, srcb_v), (1, dstb_v)):
            pltpu.make_async_copy(edge_h.at[a, pl.ds(0, GW)],
                                  arr_v.at[pl.ds(b * GW, GW)],
                                  isems[b]).wait()

    def zrow(i, t):
        for k in range(DH // 16):
            rows_v[0, i, pl.ds(16 * k, 16)] = _zero_vec()
        return t
    lax.fori_loop(0, ECH, zrow, 0)
    if first:
        def zh(i, t):
            hist_v[pl.ds(16 * i, 16)] = _zero_vec()
            return t
        lax.fori_loop(0, N_ATOMS // 16, zh, 0)

    r0 = s * STRIPE
    for k in range(5):
        pltpu.sync_copy(rows_v.at[0], acc_sh.at[pl.ds(r0 + k * ECH, ECH)])
    fire_idx(0, 0)
    drain_idx(0)
    plsc.subcore_barrier()

    def fire_gather(b, p, slot):
        return pltpu.async_copy(
            x_h.at[srcb_v.at[pl.ds(b * GW + p * ECH, ECH)]],
            rows_v.at[slot], sems[slot])

    def stage_idx(b, p, slot):
        for q in range(ECH // 16):
            w = b * GW + p * ECH + 16 * q
            sv = srcb_v[pl.ds(w, 16)]
            srcb_v[pl.ds(w, 16)] = (sv + sv + off) if first else (sv + off)
            dv = dstb_v[pl.ds(w, 16)]
            idxd_v[slot, pl.ds(16 * q, 16)] = dv
            if first:
                plsc.addupdate_scatter(hist_v, [dv],
                                       jnp.ones((16,), jnp.float32))

    def fire_scatter(slot):
        return pltpu.async_copy(rows_v.at[slot], acc_sh.at[idxd_v.at[slot]],
                                sems[slot], add=True)

    def drain_scatter(slot):
        pltpu.make_async_copy(rows_v.at[slot], acc_sh.at[idxd_v.at[slot]],
                              sems[slot]).wait()

    def process_group(b, s0, cond):
        descs = []
        for p in range(GSZ):
            @pl.when(cond)
            def _(p=p):
                drain_scatter(s0 + p)
            stage_idx(b, p, s0 + p)
            descs.append(fire_gather(b, p, s0 + p))
        for p in range(GSZ):
            descs[p].wait()
            fire_scatter(s0 + p)

    def body(i, t):
        fire_idx(2 * i + 1, 1)
        process_group(0, 0, i > 0)
        drain_idx(1)

        @pl.when(i < NGRP // 2 - 1)
        def _():
            fire_idx(2 * i + 2, 0)
        process_group(1, GSZ, i > 0)

        @pl.when(i < NGRP // 2 - 1)
        def _():
            drain_idx(0)
        return t

    lax.fori_loop(0, NGRP // 2, body, 0)
    for slot in range(2 * GSZ):
        drain_scatter(slot)

    @pl.when(s < TAIL)
    def _():
        for a, arr_v in ((0, srcb_v), (1, dstb_v)):
            pltpu.sync_copy(edge_h.at[a, pl.ds((TAIL_BASE + s) * ECH, ECH)],
                            arr_v.at[pl.ds(0, ECH)])
        stage_idx(0, 0, 0)
        fire_gather(0, 0, 0).wait()
        pltpu.sync_copy(rows_v.at[0], acc_sh.at[idxd_v.at[0]], add=True)

    if first:
        agg_h, hist_h = outs
        pltpu.sync_copy(hist_v, hist_h.at[pl.ds(wid * N_ATOMS, N_ATOMS)])
    else:
        (agg_h,) = outs

    plsc.subcore_barrier()
    for k in range(5):
        rr = r0 + k * ECH
        pltpu.sync_copy(acc_sh.at[pl.ds(rr, ECH)], rows_v.at[0])
        pltpu.sync_copy(rows_v.at[0], agg_h.at[c, pl.ds(rr, ECH)])


def _agg_first_body(edge_h, x_h, agg_h, hist_h,
                    srcb_v, dstb_v, idxd_v, rows_v, hist_v, acc_sh,
                    s0, s1, s2, s3, s4, s5, i0, i1):
    _agg_body(edge_h, x_h, (agg_h, hist_h),
              srcb_v, dstb_v, idxd_v, rows_v, hist_v, acc_sh,
              (s0, s1, s2, s3, s4, s5), (i0, i1), first=True)


def _agg_second_body(edge_h, x_h, agg_h,
                     srcb_v, dstb_v, idxd_v, rows_v, hist_v, acc_sh,
                     s0, s1, s2, s3, s4, s5, i0, i1):
    _agg_body(edge_h, x_h, (agg_h,),
              srcb_v, dstb_v, idxd_v, rows_v, hist_v, acc_sh,
              (s0, s1, s2, s3, s4, s5), (i0, i1), first=False)


def _agg_scratch():
    return [
        pltpu.VMEM((2 * GSZ * ECH,), jnp.int32),
        pltpu.VMEM((2 * GSZ * ECH,), jnp.int32),
        pltpu.VMEM((2 * GSZ, ECH), jnp.int32),
        pltpu.VMEM((2 * GSZ, ECH, DH), jnp.float32),
        pltpu.VMEM((N_ATOMS,), jnp.float32),
        pltpu.VMEM_SHARED((NAP, DH), jnp.float32),
    ] + [pltpu.SemaphoreType.DMA] * 8


_sc_params = pltpu.CompilerParams(needs_layout_passes=False,
                                  use_tc_tiling_on_sc=False)


@functools.cache
def _agg_first():
    return pl.kernel(
        _agg_first_body, mesh=_mesh(), scratch_types=_agg_scratch(),
        compiler_params=_sc_params,
        out_type=[
            jax.ShapeDtypeStruct((NC, NAP, DH), jnp.float32),
            jax.ShapeDtypeStruct((NW * N_ATOMS,), jnp.float32),
        ])


@functools.cache
def _agg_second():
    return pl.kernel(
        _agg_second_body, mesh=_mesh(), scratch_types=_agg_scratch(),
        compiler_params=_sc_params,
        out_type=jax.ShapeDtypeStruct((NC, NAP, DH), jnp.float32))


def _trip_gather_body(i0_h, i1_h, i2_h, d_h, clf_h, da_h, db_h, cl_h,
                      idx_v, rows_v, sem0, sem1, sem2, sem3):
    c = lax.axis_index("c")
    s = lax.axis_index("s")
    wid = s * NC + c
    base = wid * TPT
    sems = (sem0, sem1, sem2, sem3)
    for j, src_h in enumerate((i0_h, i1_h, i2_h)):
        pltpu.sync_copy(src_h.at[pl.ds(base, TPT)],
                        idx_v.at[pl.ds(j * TPT, TPT)])

    jobs = []
    for j, (tbl_h, out_h) in enumerate(((d_h, da_h), (d_h, db_h),
                                        (clf_h, cl_h))):
        for q in range(TPT // ECH):
            jobs.append((tbl_h, out_h, j * TPT + q * ECH, base + q * ECH))

    def fire_g(job, p):
        tbl_h, _, ioff, _ = job
        return pltpu.async_copy(tbl_h.at[idx_v.at[pl.ds(ioff, ECH)]],
                                rows_v.at[p], sems[p])

    def fire_w(job, p):
        _, out_h, _, ooff = job
        return pltpu.async_copy(rows_v.at[p], out_h.at[pl.ds(ooff, ECH)],
                                sems[p])

    def drain_w(job, p):
        _, out_h, _, ooff = job
        pltpu.make_async_copy(rows_v.at[p], out_h.at[pl.ds(ooff, ECH)],
                              sems[p]).wait()

    ngroups = len(jobs) // 4
    for g in range(ngroups):
        descs = []
        for p in range(4):
            if g > 0:
                drain_w(jobs[4 * (g - 1) + p], p)
            descs.append(fire_g(jobs[4 * g + p], p))
        for p in range(4):
            descs[p].wait()
            fire_w(jobs[4 * g + p], p)
    for p in range(4):
        drain_w(jobs[4 * (ngroups - 1) + p], p)


@functools.cache
def _trip_gather():
    return pl.kernel(
        _trip_gather_body, mesh=_mesh(),
        compiler_params=_sc_params,
        scratch_types=[
            pltpu.VMEM((3 * TPT,), jnp.int32),
            pltpu.VMEM((4, ECH, D), jnp.float32),
            pltpu.SemaphoreType.DMA,
            pltpu.SemaphoreType.DMA,
            pltpu.SemaphoreType.DMA,
            pltpu.SemaphoreType.DMA,
        ],
        out_type=[
            jax.ShapeDtypeStruct((NB, D), jnp.float32),
            jax.ShapeDtypeStruct((NB, D), jnp.float32),
            jax.ShapeDtypeStruct((NB, D), jnp.float32),
        ])


def _ln_rows(x, g, b, eps=1e-5):
    mu = jnp.mean(x, axis=-1, keepdims=True)
    var = jnp.mean((x - mu) ** 2, axis=-1, keepdims=True)
    return (x - mu) * jax.lax.rsqrt(var + eps) * g + b


def _deg_body(hist_ref, out_ref):
    deg = jnp.sum(hist_ref[...], axis=0, keepdims=True)
    out_ref[...] = 2.0 / jnp.maximum(deg, 2.0)


def _rdeg(hist):
    r = pl.pallas_call(
        _deg_body,
        out_shape=jax.ShapeDtypeStruct((1, N_ATOMS), jnp.float32),
    )(hist)
    return r.reshape(N_ATOMS, 1)


_RB0 = 2000


def _sage0_body(x_ref, agg_ref, rd_ref, ws_ref, wna_ref, wnb_ref, b_ref,
                g_ref, bl_ref, out_ref):
    rd = rd_ref[...]
    y = (jnp.dot(x_ref[...], ws_ref[...], preferred_element_type=jnp.float32)
         + jnp.dot(agg_ref[0] * rd, wna_ref[...],
                   preferred_element_type=jnp.float32)
         + jnp.dot(agg_ref[1] * rd, wnb_ref[...],
                   preferred_element_type=jnp.float32)
         + b_ref[...])
    h = jnp.maximum(_ln_rows(y, g_ref[...], bl_ref[...]), 0.0)
    out_ref[0] = h[:, :DH]
    out_ref[1] = h[:, DH:]


def _sage0(x, agg2, rdeg, Ws, Wna, Wnb, b, g, bl):
    n = N_ATOMS // _RB0
    return pl.pallas_call(
        _sage0_body,
        grid=(n,),
        in_specs=[
            pl.BlockSpec((_RB0, D), lambda i: (i, 0)),
            pl.BlockSpec((NC, _RB0, DH), lambda i: (0, i, 0)),
            pl.BlockSpec((_RB0, 1), lambda i: (i, 0)),
            pl.BlockSpec((D, D), lambda i: (0, 0)),
            pl.BlockSpec((DH, D), lambda i: (0, 0)),
            pl.BlockSpec((DH, D), lambda i: (0, 0)),
            pl.BlockSpec((1, D), lambda i: (0, 0)),
            pl.BlockSpec((1, D), lambda i: (0, 0)),
            pl.BlockSpec((1, D), lambda i: (0, 0)),
        ],
        out_specs=pl.BlockSpec((NC, _RB0, DH), lambda i: (0, i, 0)),
        out_shape=jax.ShapeDtypeStruct((NC, N_ATOMS, DH), jnp.float32),
    )(x, agg2, rdeg, Ws, Wna, Wnb, b, g, bl)


def _sage1_body(h_ref, agg_ref, rd_ref, wsa_ref, wsb_ref, wna_ref, wnb_ref,
                b_ref, wo_ref, bo_ref, g_ref, bl_ref, out_ref):
    rd = rd_ref[...]
    y = (jnp.dot(h_ref[0], wsa_ref[...], preferred_element_type=jnp.float32)
         + jnp.dot(h_ref[1], wsb_ref[...], preferred_element_type=jnp.float32)
         + jnp.dot(agg_ref[0] * rd, wna_ref[...],
                   preferred_element_type=jnp.float32)
         + jnp.dot(agg_ref[1] * rd, wnb_ref[...],
                   preferred_element_type=jnp.float32)
         + b_ref[...])
    pooled = jnp.max(y.reshape(_RB0 // 10, 10, D), axis=1)
    dd = jnp.maximum(
        jnp.dot(pooled, wo_ref[...], preferred_element_type=jnp.float32)
        + bo_ref[...], 0.0)
    out_ref[...] = _ln_rows(dd, g_ref[...], bl_ref[...])


def _sage1(h2, agg2, rdeg, Wsa, Wsb, Wna, Wnb, b, Wo, bo, g, bl):
    n = N_ATOMS // _RB0
    return pl.pallas_call(
        _sage1_body,
        grid=(n,),
        in_specs=[
            pl.BlockSpec((NC, _RB0, DH), lambda i: (0, i, 0)),
            pl.BlockSpec((NC, _RB0, DH), lambda i: (0, i, 0)),
            pl.BlockSpec((_RB0, 1), lambda i: (i, 0)),
            pl.BlockSpec((DH, D), lambda i: (0, 0)),
            pl.BlockSpec((DH, D), lambda i: (0, 0)),
            pl.BlockSpec((DH, D), lambda i: (0, 0)),
            pl.BlockSpec((DH, D), lambda i: (0, 0)),
            pl.BlockSpec((1, D), lambda i: (0, 0)),
            pl.BlockSpec((D, D), lambda i: (0, 0)),
            pl.BlockSpec((1, D), lambda i: (0, 0)),
            pl.BlockSpec((1, D), lambda i: (0, 0)),
            pl.BlockSpec((1, D), lambda i: (0, 0)),
        ],
        out_specs=pl.BlockSpec((_RB0 // 10, D), lambda i: (i, 0)),
        out_shape=jax.ShapeDtypeStruct((N_DRUGS, D), jnp.float32),
    )(h2, agg2, rdeg, Wsa, Wsb, Wna, Wnb, b, Wo, bo, g, bl)


_RBF = 2048


def _mlp_body(da_ref, db_ref, cl_ref, gcl_ref, bcl_ref,
              w0a_ref, w0b_ref, w0c_ref, bt0_ref, gt_ref, btl_ref,
              wt1_ref, bt1_ref, wp_ref, bp_ref, out_ref):
    cl = _ln_rows(cl_ref[...], gcl_ref[...], bcl_ref[...])
    t = (jnp.dot(da_ref[...], w0a_ref[...], preferred_element_type=jnp.float32)
         + jnp.dot(db_ref[...], w0b_ref[...], preferred_element_type=jnp.float32)
         + jnp.dot(cl, w0c_ref[...], preferred_element_type=jnp.float32)
         + bt0_ref[...])
    h = _ln_rows(jnp.maximum(t, 0.0), gt_ref[...], btl_ref[...])
    h2 = jnp.maximum(
        jnp.dot(h, wt1_ref[...], preferred_element_type=jnp.float32)
        + bt1_ref[...], 0.0)
    out_ref[...] = (jnp.dot(h2, wp_ref[...],
                            preferred_element_type=jnp.float32) + bp_ref[...])


def _mlp(da, db, cl, gcl, bcl, W0a, W0b, W0c, bt0, gt, btl, Wt1, bt1, Wp, bp):
    n = NB // _RBF
    row = lambda i: (i, 0)
    fix = lambda i: (0, 0)
    return pl.pallas_call(
        _mlp_body,
        grid=(n,),
        in_specs=[
            pl.BlockSpec((_RBF, D), row),
            pl.BlockSpec((_RBF, D), row),
            pl.BlockSpec((_RBF, D), row),
            pl.BlockSpec((1, D), fix),
            pl.BlockSpec((1, D), fix),
            pl.BlockSpec((D, H0), fix),
            pl.BlockSpec((D, H0), fix),
            pl.BlockSpec((D, H0), fix),
            pl.BlockSpec((1, H0), fix),
            pl.BlockSpec((1, H0), fix),
            pl.BlockSpec((1, H0), fix),
            pl.BlockSpec((H0, H1), fix),
            pl.BlockSpec((1, H1), fix),
            pl.BlockSpec((H1, 1), fix),
            pl.BlockSpec((1, 1), fix),
        ],
        out_specs=pl.BlockSpec((_RBF, 1), row),
        out_shape=jax.ShapeDtypeStruct((NB, 1), jnp.float32),
    )(da, db, cl, gcl, bcl, W0a, W0b, W0c, bt0, gt, btl, Wt1, bt1, Wp, bp)


def kernel(indices, atom_features, edge_index, batch, Ws0, Wn0, b0, g_ln0,
           b_ln0, Ws1, Wn1, b1, Wout, b_out, g_dn, b_dn, cl_feat, g_cl, b_cl,
           W_t0, b_t0, g_t, b_tln, W_t1, b_t1, W_p, b_p):
    idx0 = indices[:, 0]
    idx1 = indices[:, 1]
    idx2 = indices[:, 2]
    r1 = lambda v: v.reshape(1, -1)

    agg0, hist = _agg_first()(edge_index,
                              atom_features.reshape(NC * N_ATOMS, DH))
    rdeg = _rdeg(hist.reshape(NW, N_ATOMS))
    h2 = _sage0(atom_features, agg0, rdeg, Ws0, Wn0[:DH], Wn0[DH:], r1(b0),
                r1(g_ln0), r1(b_ln0))
    agg1 = _agg_second()(edge_index, h2.reshape(NC * N_ATOMS, DH))
    d = _sage1(h2, agg1, rdeg, Ws1[:DH], Ws1[DH:], Wn1[:DH], Wn1[DH:],
               r1(b1), Wout, r1(b_out), r1(g_dn), r1(b_dn))
    da, db, cl_rows = _trip_gather()(idx0, idx1, idx2, d, cl_feat)
    out = _mlp(da, db, cl_rows, r1(g_cl), r1(b_cl),
               W_t0[:D], W_t0[D:2 * D], W_t0[2 * D:], r1(b_t0), r1(g_t),
               r1(b_tln), W_t1, r1(b_t1), W_p, r1(b_p))
    return out[:, 0]
